# Initial kernel scaffold; baseline (speedup 1.0000x reference)
import jax
import jax.numpy as jnp
from jax.experimental import pallas as pl


def kernel(hidden_states, gate_w, gate_b, Wg, Wu, Wd, Sg, Su, Sd):
    raise NotImplementedError("write your pallas kernel here")


# speedup vs baseline: 8.3625x; 8.3625x over previous
import functools

import jax
import jax.numpy as jnp
from jax import lax
from jax.experimental import pallas as pl
from jax.experimental.pallas import tpu as pltpu
from jax.experimental.pallas import tpu_sc as plsc

HB = 1024
EE = 64
KK = 2
NGG = 8
TGG = 4
DF = 512
WSCALE = 2.5
MT = 128
NT = 96
NPAD = NT * MT
NTOK = 2048
NPAIR = NTOK * KK


def _gate_route_body(x_ref, gw_ref, gb_ref, w_out, dest_out, te_out):
    x = x_ref[...]
    gw = gw_ref[...]
    gb = gb_ref[...]
    logits = lax.dot_general(x, gw, (((1,), (1,)), ((), ())),
                             preferred_element_type=jnp.float32)
    scores = jax.nn.sigmoid(logits)
    sfc = scores + gb
    n = x.shape[0]
    neg = jnp.float32(-jnp.inf)
    iota8 = lax.broadcasted_iota(jnp.int32, (n, NGG), 1)

    gparts = []
    for g in range(NGG):
        sub = sfc[:, g * 8:(g + 1) * 8]
        m1 = jnp.max(sub, axis=-1, keepdims=True)
        j1 = jnp.min(jnp.where(sub == m1, iota8, NGG), axis=-1, keepdims=True)
        m2 = jnp.max(jnp.where(iota8 == j1, neg, sub), axis=-1, keepdims=True)
        gparts.append(m1 + m2)
    gs = jnp.concatenate(gparts, axis=1)

    gmask = jnp.zeros((n, NGG), jnp.float32)
    cur = gs
    for _ in range(TGG):
        m = jnp.max(cur, axis=-1, keepdims=True)
        j = jnp.min(jnp.where(cur == m, iota8, NGG), axis=-1, keepdims=True)
        hit = iota8 == j
        gmask = gmask + hit.astype(jnp.float32)
        cur = jnp.where(hit, neg, cur)

    er = lax.broadcasted_iota(jnp.int32, (NGG, EE), 0)
    ec = lax.broadcasted_iota(jnp.int32, (NGG, EE), 1)
    expand = (er == ec // NGG).astype(jnp.float32)
    smask = lax.dot_general(gmask, expand, (((1,), (0,)), ((), ())),
                            preferred_element_type=jnp.float32)

    iota64 = lax.broadcasted_iota(jnp.int32, (n, EE), 1)
    tmp = jnp.where(smask > 0, sfc, neg)
    ohs, ws = [], []
    for _ in range(KK):
        m = jnp.max(tmp, axis=-1, keepdims=True)
        j = jnp.min(jnp.where(tmp == m, iota64, EE), axis=-1, keepdims=True)
        oh = (iota64 == j).astype(jnp.float32)
        ohs.append(oh)
        ws.append(jnp.sum(oh * scores, axis=-1, keepdims=True))
        tmp = jnp.where(oh > 0, neg, tmp)
    w0, w1 = ws
    denom = w0 + w1 + 1e-20
    w_out[...] = jnp.concatenate(
        [w0 / denom * WSCALE, w1 / denom * WSCALE], axis=1)

    oh_all = ohs[0] + ohs[1]
    counts = jnp.sum(oh_all, axis=0, keepdims=True)
    ptiles = jnp.floor((counts + (MT - 1)) * (1.0 / MT))
    ur = lax.broadcasted_iota(jnp.int32, (EE, EE), 0)
    uc = lax.broadcasted_iota(jnp.int32, (EE, EE), 1)
    strict_u = (ur < uc).astype(jnp.float32)
    tstart = lax.dot_general(ptiles, strict_u, (((1,), (0,)), ((), ())),
                             preferred_element_type=jnp.float32)
    tstart_i = tstart.astype(jnp.int32)
    trow = lax.broadcasted_iota(jnp.int32, (NT, EE), 0)
    te = jnp.sum((jnp.broadcast_to(tstart_i, (NT, EE)) <= trow)
                 .astype(jnp.int32), axis=-1, keepdims=True) - 1
    te_out[...] = te

    blk = 256
    br = lax.broadcasted_iota(jnp.int32, (blk, blk), 0)
    bc = lax.broadcasted_iota(jnp.int32, (blk, blk), 1)
    lstrict = (bc < br).astype(jnp.float32)
    tot = jnp.zeros((1, EE), jnp.float32)
    cparts = []
    for b in range(n // blk):
        ohb = oh_all[b * blk:(b + 1) * blk, :]
        cb = lax.dot_general(lstrict, ohb, (((1,), (0,)), ((), ())),
                             preferred_element_type=jnp.float32) + tot
        cparts.append(cb)
        tot = tot + jnp.sum(ohb, axis=0, keepdims=True)
    cexc = jnp.concatenate(cparts, axis=0)

    rowbase = tstart * MT + cexc
    d0 = jnp.sum(ohs[0] * rowbase, axis=-1, keepdims=True)
    d1 = jnp.sum(ohs[1] * rowbase, axis=-1, keepdims=True)
    dest_out[...] = jnp.concatenate([d0, d1], axis=1).astype(jnp.int32)


def _moe_body(te_ref, xs_ref, wg_ref, wu_ref, wd_ref, out_ref):
    xb = xs_ref[...]
    wg = wg_ref[0]
    wu = wu_ref[0]
    wd = wd_ref[0]
    a = lax.dot_general(xb, wg, (((1,), (1,)), ((), ())),
                        preferred_element_type=jnp.float32)
    u = lax.dot_general(xb, wu, (((1,), (1,)), ((), ())),
                        preferred_element_type=jnp.float32)
    h = a * jax.nn.sigmoid(a) * u
    out_ref[...] = lax.dot_general(h, wd, (((1,), (1,)), ((), ())),
                                   preferred_element_type=jnp.float32)


def _combine_body(x_ref, sg_ref, su_ref, sd_ref, gp_ref, w_ref, out_ref):
    xb = x_ref[...]
    a = lax.dot_general(xb, sg_ref[...], (((1,), (1,)), ((), ())),
                        preferred_element_type=jnp.float32)
    u = lax.dot_general(xb, su_ref[...], (((1,), (1,)), ((), ())),
                        preferred_element_type=jnp.float32)
    h = a * jax.nn.sigmoid(a) * u
    sh = lax.dot_general(h, sd_ref[...], (((1,), (1,)), ((), ())),
                         preferred_element_type=jnp.float32)
    gp = gp_ref[...]
    ge = gp[:, :HB]
    go = gp[:, HB:]
    w = w_ref[...]
    out_ref[...] = sh + w[:, 0:1] * ge + w[:, 1:2] * go


def _sc_dims():
    try:
        info = plsc.get_sparse_core_info()
        return info.num_cores, info.num_subcores
    except Exception:
        return 2, 16


def _make_dispatch(nc, ns):
    nw = nc * ns
    bpw = NPAIR // nw
    ch = 32
    mesh = plsc.VectorSubcoreMesh(core_axis_name="c", subcore_axis_name="s")

    @functools.partial(
        pl.kernel, mesh=mesh,
        out_type=jax.ShapeDtypeStruct((NPAD, HB), jnp.float32),
        scratch_types=[
            pltpu.VMEM((ch,), jnp.int32),
            pltpu.VMEM((ch,), jnp.int32),
            pltpu.VMEM((ch, HB), jnp.float32),
            pltpu.SemaphoreType.DMA,
        ],
    )
    def dispatch(x_hbm, dest_hbm, xs_hbm, src_v, dst_v, rows_v, sem):
        wid = lax.axis_index("s") * nc + lax.axis_index("c")
        base = wid * bpw

        def chunk(c, carry):
            off = base + c * ch
            for v in range(ch // 16):
                io = lax.iota(jnp.int32, 16)
                src_v[pl.ds(v * 16, 16)] = lax.shift_right_logical(
                    off + v * 16 + io, 1)
            pltpu.sync_copy(dest_hbm.at[pl.ds(off, ch)], dst_v)
            pltpu.async_copy(x_hbm.at[src_v], rows_v, sem).wait()
            pltpu.async_copy(rows_v, xs_hbm.at[dst_v], sem).wait()
            return carry

        lax.fori_loop(0, bpw // ch, chunk, 0)

    return dispatch


def _make_gather(nc, ns):
    nw = nc * ns
    bpw = NPAIR // nw
    ch = 32
    mesh = plsc.VectorSubcoreMesh(core_axis_name="c", subcore_axis_name="s")

    @functools.partial(
        pl.kernel, mesh=mesh,
        out_type=jax.ShapeDtypeStruct((NPAIR, HB), jnp.float32),
        scratch_types=[
            pltpu.VMEM((ch,), jnp.int32),
            pltpu.VMEM((ch, HB), jnp.float32),
            pltpu.SemaphoreType.DMA,
        ],
    )
    def gather(ys_hbm, dest_hbm, gp_hbm, idx_v, rows_v, sem):
        wid = lax.axis_index("s") * nc + lax.axis_index("c")
        base = wid * bpw

        def chunk(c, carry):
            off = base + c * ch
            pltpu.sync_copy(dest_hbm.at[pl.ds(off, ch)], idx_v)
            pltpu.async_copy(ys_hbm.at[idx_v], rows_v, sem).wait()
            pltpu.sync_copy(rows_v, gp_hbm.at[pl.ds(off, ch)])
            return carry

        lax.fori_loop(0, bpw // ch, chunk, 0)

    return gather


def kernel(hidden_states, gate_w, gate_b, Wg, Wu, Wd, Sg, Su, Sd):
    bsz, seq, hid = hidden_states.shape
    x = hidden_states.reshape(-1, hid)

    topk_w, dest, te = pl.pallas_call(
        _gate_route_body,
        out_shape=(
            jax.ShapeDtypeStruct((NTOK, KK), jnp.float32),
            jax.ShapeDtypeStruct((NTOK, KK), jnp.int32),
            jax.ShapeDtypeStruct((NT, 1), jnp.int32),
        ),
    )(x, gate_w, gate_b.reshape(1, EE))
    dest_flat = dest.reshape(-1)
    te_flat = te.reshape(-1)

    nc, ns = _sc_dims()

    xs = _make_dispatch(nc, ns)(x, dest_flat)

    grid_spec = pltpu.PrefetchScalarGridSpec(
        num_scalar_prefetch=1,
        grid=(NT,),
        in_specs=[
            pl.BlockSpec((MT, HB), lambda i, te_r: (i, 0)),
            pl.BlockSpec((1, DF, HB), lambda i, te_r: (te_r[i], 0, 0)),
            pl.BlockSpec((1, DF, HB), lambda i, te_r: (te_r[i], 0, 0)),
            pl.BlockSpec((1, HB, DF), lambda i, te_r: (te_r[i], 0, 0)),
        ],
        out_specs=pl.BlockSpec((MT, HB), lambda i, te_r: (i, 0)),
    )
    ys = pl.pallas_call(
        _moe_body,
        grid_spec=grid_spec,
        out_shape=jax.ShapeDtypeStruct((NPAD, HB), jnp.float32),
    )(te_flat, xs, Wg, Wu, Wd)

    gp = _make_gather(nc, ns)(ys, dest_flat)
    gp2 = gp.reshape(NTOK, KK * HB)

    cb = 256
    y = pl.pallas_call(
        _combine_body,
        grid=(NTOK // cb,),
        in_specs=[
            pl.BlockSpec((cb, HB), lambda i: (i, 0)),
            pl.BlockSpec((DF, HB), lambda i: (0, 0)),
            pl.BlockSpec((DF, HB), lambda i: (0, 0)),
            pl.BlockSpec((HB, DF), lambda i: (0, 0)),
            pl.BlockSpec((cb, KK * HB), lambda i: (i, 0)),
            pl.BlockSpec((cb, KK), lambda i: (i, 0)),
        ],
        out_specs=pl.BlockSpec((cb, HB), lambda i: (i, 0)),
        out_shape=jax.ShapeDtypeStruct((NTOK, HB), jnp.float32),
    )(x, Sg, Su, Sd, gp2, topk_w)

    return y.reshape(bsz, seq, hid)


# speedup vs baseline: 8.8972x; 1.0639x over previous
import functools

import jax
import jax.numpy as jnp
from jax import lax
from jax.experimental import pallas as pl
from jax.experimental.pallas import tpu as pltpu
from jax.experimental.pallas import tpu_sc as plsc

HB = 1024
EE = 64
KK = 2
NGG = 8
TGG = 4
DF = 512
WSCALE = 2.5
MT = 128
NT = 96
NPAD = NT * MT
NTOK = 2048
NPAIR = NTOK * KK


def _gate_route_body(x_ref, gw_ref, gb_ref, w_out, dest_out, te_out):
    x = x_ref[...]
    gw = gw_ref[...]
    gb = gb_ref[...]
    logits = lax.dot_general(x, gw, (((1,), (1,)), ((), ())),
                             preferred_element_type=jnp.float32)
    scores = jax.nn.sigmoid(logits)
    sfc = scores + gb
    n = x.shape[0]
    neg = jnp.float32(-jnp.inf)
    iota8 = lax.broadcasted_iota(jnp.int32, (n, NGG), 1)

    gparts = []
    for g in range(NGG):
        sub = sfc[:, g * 8:(g + 1) * 8]
        m1 = jnp.max(sub, axis=-1, keepdims=True)
        j1 = jnp.min(jnp.where(sub == m1, iota8, NGG), axis=-1, keepdims=True)
        m2 = jnp.max(jnp.where(iota8 == j1, neg, sub), axis=-1, keepdims=True)
        gparts.append(m1 + m2)
    gs = jnp.concatenate(gparts, axis=1)

    gmask = jnp.zeros((n, NGG), jnp.float32)
    cur = gs
    for _ in range(TGG):
        m = jnp.max(cur, axis=-1, keepdims=True)
        j = jnp.min(jnp.where(cur == m, iota8, NGG), axis=-1, keepdims=True)
        hit = iota8 == j
        gmask = gmask + hit.astype(jnp.float32)
        cur = jnp.where(hit, neg, cur)

    er = lax.broadcasted_iota(jnp.int32, (NGG, EE), 0)
    ec = lax.broadcasted_iota(jnp.int32, (NGG, EE), 1)
    expand = (er == ec // NGG).astype(jnp.float32)
    smask = lax.dot_general(gmask, expand, (((1,), (0,)), ((), ())),
                            preferred_element_type=jnp.float32)

    iota64 = lax.broadcasted_iota(jnp.int32, (n, EE), 1)
    tmp = jnp.where(smask > 0, sfc, neg)
    ohs, ws = [], []
    for _ in range(KK):
        m = jnp.max(tmp, axis=-1, keepdims=True)
        j = jnp.min(jnp.where(tmp == m, iota64, EE), axis=-1, keepdims=True)
        oh = (iota64 == j).astype(jnp.float32)
        ohs.append(oh)
        ws.append(jnp.sum(oh * scores, axis=-1, keepdims=True))
        tmp = jnp.where(oh > 0, neg, tmp)
    w0, w1 = ws
    denom = w0 + w1 + 1e-20
    w_out[...] = jnp.concatenate(
        [w0 / denom * WSCALE, w1 / denom * WSCALE], axis=1)

    oh_all = ohs[0] + ohs[1]
    counts = jnp.sum(oh_all, axis=0, keepdims=True)
    ptiles = jnp.floor((counts + (MT - 1)) * (1.0 / MT))
    ur = lax.broadcasted_iota(jnp.int32, (EE, EE), 0)
    uc = lax.broadcasted_iota(jnp.int32, (EE, EE), 1)
    strict_u = (ur < uc).astype(jnp.float32)
    tstart = lax.dot_general(ptiles, strict_u, (((1,), (0,)), ((), ())),
                             preferred_element_type=jnp.float32)
    tstart_i = tstart.astype(jnp.int32)
    trow = lax.broadcasted_iota(jnp.int32, (NT + 1, EE), 0)
    te = jnp.sum((jnp.broadcast_to(tstart_i, (NT + 1, EE)) <= trow)
                 .astype(jnp.int32), axis=-1, keepdims=True) - 1
    used = jnp.sum(ptiles, axis=-1, keepdims=True).astype(jnp.int32)
    te_out[...] = jnp.concatenate([te[:NT], used], axis=0)

    blk = 256
    br = lax.broadcasted_iota(jnp.int32, (blk, blk), 0)
    bc = lax.broadcasted_iota(jnp.int32, (blk, blk), 1)
    lstrict = (bc < br).astype(jnp.float32)
    tot = jnp.zeros((1, EE), jnp.float32)
    cparts = []
    for b in range(n // blk):
        ohb = oh_all[b * blk:(b + 1) * blk, :]
        cb = lax.dot_general(lstrict, ohb, (((1,), (0,)), ((), ())),
                             preferred_element_type=jnp.float32) + tot
        cparts.append(cb)
        tot = tot + jnp.sum(ohb, axis=0, keepdims=True)
    cexc = jnp.concatenate(cparts, axis=0)

    rowbase = tstart * MT + cexc
    d0 = jnp.sum(ohs[0] * rowbase, axis=-1, keepdims=True)
    d1 = jnp.sum(ohs[1] * rowbase, axis=-1, keepdims=True)
    dest_out[...] = jnp.concatenate([d0, d1], axis=1).astype(jnp.int32)


def _moe_body(te_ref, xs_ref, wg_ref, wu_ref, wd_ref, out_ref):
    @pl.when(pl.program_id(0) < te_ref[NT])
    def _():
        xb = xs_ref[...].astype(jnp.bfloat16)
        wg = wg_ref[0].astype(jnp.bfloat16)
        wu = wu_ref[0].astype(jnp.bfloat16)
        wd = wd_ref[0].astype(jnp.bfloat16)
        a = lax.dot_general(xb, wg, (((1,), (1,)), ((), ())),
                            preferred_element_type=jnp.float32)
        u = lax.dot_general(xb, wu, (((1,), (1,)), ((), ())),
                            preferred_element_type=jnp.float32)
        h = (a * jax.nn.sigmoid(a) * u).astype(jnp.bfloat16)
        out_ref[...] = lax.dot_general(h, wd, (((1,), (1,)), ((), ())),
                                       preferred_element_type=jnp.float32)


def _combine_body(x_ref, sg_ref, su_ref, sd_ref, gp_ref, w_ref, out_ref):
    xb = x_ref[...].astype(jnp.bfloat16)
    a = lax.dot_general(xb, sg_ref[...].astype(jnp.bfloat16),
                        (((1,), (1,)), ((), ())),
                        preferred_element_type=jnp.float32)
    u = lax.dot_general(xb, su_ref[...].astype(jnp.bfloat16),
                        (((1,), (1,)), ((), ())),
                        preferred_element_type=jnp.float32)
    h = (a * jax.nn.sigmoid(a) * u).astype(jnp.bfloat16)
    sh = lax.dot_general(h, sd_ref[...].astype(jnp.bfloat16),
                         (((1,), (1,)), ((), ())),
                         preferred_element_type=jnp.float32)
    gp = gp_ref[...]
    ge = gp[:, :HB]
    go = gp[:, HB:]
    w = w_ref[...]
    out_ref[...] = sh + w[:, 0:1] * ge + w[:, 1:2] * go


def _sc_dims():
    try:
        info = plsc.get_sparse_core_info()
        return info.num_cores, info.num_subcores
    except Exception:
        return 2, 16


def _make_dispatch(nc, ns):
    nw = nc * ns
    bpw = NPAIR // nw
    ch = 32
    mesh = plsc.VectorSubcoreMesh(core_axis_name="c", subcore_axis_name="s")

    @functools.partial(
        pl.kernel, mesh=mesh,
        out_type=jax.ShapeDtypeStruct((NPAD, HB), jnp.float32),
        scratch_types=[
            pltpu.VMEM((ch,), jnp.int32),
            pltpu.VMEM((ch,), jnp.int32),
            pltpu.VMEM((ch, HB), jnp.float32),
            pltpu.SemaphoreType.DMA,
        ],
    )
    def dispatch(x_hbm, dest_hbm, xs_hbm, src_v, dst_v, rows_v, sem):
        wid = lax.axis_index("s") * nc + lax.axis_index("c")
        base = wid * bpw

        def chunk(c, carry):
            off = base + c * ch
            for v in range(ch // 16):
                io = lax.iota(jnp.int32, 16)
                src_v[pl.ds(v * 16, 16)] = lax.shift_right_logical(
                    off + v * 16 + io, 1)
            pltpu.sync_copy(dest_hbm.at[pl.ds(off, ch)], dst_v)
            pltpu.async_copy(x_hbm.at[src_v], rows_v, sem).wait()
            pltpu.async_copy(rows_v, xs_hbm.at[dst_v], sem).wait()
            return carry

        lax.fori_loop(0, bpw // ch, chunk, 0)

    return dispatch


def _make_gather(nc, ns):
    nw = nc * ns
    bpw = NPAIR // nw
    ch = 32
    mesh = plsc.VectorSubcoreMesh(core_axis_name="c", subcore_axis_name="s")

    @functools.partial(
        pl.kernel, mesh=mesh,
        out_type=jax.ShapeDtypeStruct((NPAIR, HB), jnp.float32),
        scratch_types=[
            pltpu.VMEM((ch,), jnp.int32),
            pltpu.VMEM((ch, HB), jnp.float32),
            pltpu.SemaphoreType.DMA,
        ],
    )
    def gather(ys_hbm, dest_hbm, gp_hbm, idx_v, rows_v, sem):
        wid = lax.axis_index("s") * nc + lax.axis_index("c")
        base = wid * bpw

        def chunk(c, carry):
            off = base + c * ch
            pltpu.sync_copy(dest_hbm.at[pl.ds(off, ch)], idx_v)
            pltpu.async_copy(ys_hbm.at[idx_v], rows_v, sem).wait()
            pltpu.sync_copy(rows_v, gp_hbm.at[pl.ds(off, ch)])
            return carry

        lax.fori_loop(0, bpw // ch, chunk, 0)

    return gather


def kernel(hidden_states, gate_w, gate_b, Wg, Wu, Wd, Sg, Su, Sd):
    bsz, seq, hid = hidden_states.shape
    x = hidden_states.reshape(-1, hid)

    topk_w, dest, te = pl.pallas_call(
        _gate_route_body,
        out_shape=(
            jax.ShapeDtypeStruct((NTOK, KK), jnp.float32),
            jax.ShapeDtypeStruct((NTOK, KK), jnp.int32),
            jax.ShapeDtypeStruct((NT + 1, 1), jnp.int32),
        ),
    )(x, gate_w, gate_b.reshape(1, EE))
    dest_flat = dest.reshape(-1)
    te_flat = te.reshape(-1)

    nc, ns = _sc_dims()

    xs = _make_dispatch(nc, ns)(x, dest_flat)

    grid_spec = pltpu.PrefetchScalarGridSpec(
        num_scalar_prefetch=1,
        grid=(NT,),
        in_specs=[
            pl.BlockSpec((MT, HB), lambda i, te_r: (i, 0)),
            pl.BlockSpec((1, DF, HB), lambda i, te_r: (te_r[i], 0, 0)),
            pl.BlockSpec((1, DF, HB), lambda i, te_r: (te_r[i], 0, 0)),
            pl.BlockSpec((1, HB, DF), lambda i, te_r: (te_r[i], 0, 0)),
        ],
        out_specs=pl.BlockSpec((MT, HB), lambda i, te_r: (i, 0)),
    )
    ys = pl.pallas_call(
        _moe_body,
        grid_spec=grid_spec,
        out_shape=jax.ShapeDtypeStruct((NPAD, HB), jnp.float32),
    )(te_flat, xs, Wg, Wu, Wd)

    gp = _make_gather(nc, ns)(ys, dest_flat)
    gp2 = gp.reshape(NTOK, KK * HB)

    cb = 256
    y = pl.pallas_call(
        _combine_body,
        grid=(NTOK // cb,),
        in_specs=[
            pl.BlockSpec((cb, HB), lambda i: (i, 0)),
            pl.BlockSpec((DF, HB), lambda i: (0, 0)),
            pl.BlockSpec((DF, HB), lambda i: (0, 0)),
            pl.BlockSpec((HB, DF), lambda i: (0, 0)),
            pl.BlockSpec((cb, KK * HB), lambda i: (i, 0)),
            pl.BlockSpec((cb, KK), lambda i: (i, 0)),
        ],
        out_specs=pl.BlockSpec((cb, HB), lambda i: (i, 0)),
        out_shape=jax.ShapeDtypeStruct((NTOK, HB), jnp.float32),
    )(x, Sg, Su, Sd, gp2, topk_w)

    return y.reshape(bsz, seq, hid)


# speedup vs baseline: 9.0339x; 1.0154x over previous
import functools

import jax
import jax.numpy as jnp
from jax import lax
from jax.experimental import pallas as pl
from jax.experimental.pallas import tpu as pltpu
from jax.experimental.pallas import tpu_sc as plsc

HB = 1024
EE = 64
KK = 2
NGG = 8
TGG = 4
DF = 512
WSCALE = 2.5
MT = 128
NT = 96
NPAD = NT * MT
NTOK = 2048
NPAIR = NTOK * KK


def _gate_route_body(x_ref, gw_ref, gb_ref, w_out, dest_out, te_out):
    x = x_ref[...]
    gw = gw_ref[...]
    gb = gb_ref[...]
    logits = lax.dot_general(x, gw, (((1,), (1,)), ((), ())),
                             preferred_element_type=jnp.float32)
    scores = jax.nn.sigmoid(logits)
    sfc = scores + gb
    n = x.shape[0]
    neg = jnp.float32(-jnp.inf)
    iota8 = lax.broadcasted_iota(jnp.int32, (n, NGG), 1)

    gparts = []
    for g in range(NGG):
        sub = sfc[:, g * 8:(g + 1) * 8]
        m1 = jnp.max(sub, axis=-1, keepdims=True)
        j1 = jnp.min(jnp.where(sub == m1, iota8, NGG), axis=-1, keepdims=True)
        m2 = jnp.max(jnp.where(iota8 == j1, neg, sub), axis=-1, keepdims=True)
        gparts.append(m1 + m2)
    gs = jnp.concatenate(gparts, axis=1)

    gmask = jnp.zeros((n, NGG), jnp.float32)
    cur = gs
    for _ in range(TGG):
        m = jnp.max(cur, axis=-1, keepdims=True)
        j = jnp.min(jnp.where(cur == m, iota8, NGG), axis=-1, keepdims=True)
        hit = iota8 == j
        gmask = gmask + hit.astype(jnp.float32)
        cur = jnp.where(hit, neg, cur)

    er = lax.broadcasted_iota(jnp.int32, (NGG, EE), 0)
    ec = lax.broadcasted_iota(jnp.int32, (NGG, EE), 1)
    expand = (er == ec // NGG).astype(jnp.float32)
    smask = lax.dot_general(gmask, expand, (((1,), (0,)), ((), ())),
                            preferred_element_type=jnp.float32)

    iota64 = lax.broadcasted_iota(jnp.int32, (n, EE), 1)
    tmp = jnp.where(smask > 0, sfc, neg)
    ohs, ws = [], []
    for _ in range(KK):
        m = jnp.max(tmp, axis=-1, keepdims=True)
        j = jnp.min(jnp.where(tmp == m, iota64, EE), axis=-1, keepdims=True)
        oh = (iota64 == j).astype(jnp.float32)
        ohs.append(oh)
        ws.append(jnp.sum(oh * scores, axis=-1, keepdims=True))
        tmp = jnp.where(oh > 0, neg, tmp)
    w0, w1 = ws
    denom = w0 + w1 + 1e-20
    w_out[...] = jnp.concatenate(
        [w0 / denom * WSCALE, w1 / denom * WSCALE], axis=1)

    oh_all = ohs[0] + ohs[1]
    counts = jnp.sum(oh_all, axis=0, keepdims=True)
    ptiles = jnp.floor((counts + (MT - 1)) * (1.0 / MT))
    ur = lax.broadcasted_iota(jnp.int32, (EE, EE), 0)
    uc = lax.broadcasted_iota(jnp.int32, (EE, EE), 1)
    strict_u = (ur < uc).astype(jnp.float32)
    tstart = lax.dot_general(ptiles, strict_u, (((1,), (0,)), ((), ())),
                             preferred_element_type=jnp.float32)
    tstart_i = tstart.astype(jnp.int32)
    trow = lax.broadcasted_iota(jnp.int32, (NT + 1, EE), 0)
    te = jnp.sum((jnp.broadcast_to(tstart_i, (NT + 1, EE)) <= trow)
                 .astype(jnp.int32), axis=-1, keepdims=True) - 1
    used = jnp.sum(ptiles, axis=-1, keepdims=True).astype(jnp.int32)
    te_out[...] = jnp.concatenate([te[:NT], used], axis=0)

    blk = 256
    br = lax.broadcasted_iota(jnp.int32, (blk, blk), 0)
    bc = lax.broadcasted_iota(jnp.int32, (blk, blk), 1)
    lstrict = (bc < br).astype(jnp.float32)
    tot = jnp.zeros((1, EE), jnp.float32)
    cparts = []
    for b in range(n // blk):
        ohb = oh_all[b * blk:(b + 1) * blk, :]
        cb = lax.dot_general(lstrict, ohb, (((1,), (0,)), ((), ())),
                             preferred_element_type=jnp.float32) + tot
        cparts.append(cb)
        tot = tot + jnp.sum(ohb, axis=0, keepdims=True)
    cexc = jnp.concatenate(cparts, axis=0)

    rowbase = tstart * MT + cexc
    d0 = jnp.sum(ohs[0] * rowbase, axis=-1, keepdims=True)
    d1 = jnp.sum(ohs[1] * rowbase, axis=-1, keepdims=True)
    dest_out[...] = jnp.concatenate([d0, d1], axis=1).astype(jnp.int32)


def _moe_body(te_ref, xs_ref, wg_ref, wu_ref, wd_ref, out_ref):
    @pl.when(pl.program_id(0) < te_ref[NT])
    def _():
        xb = xs_ref[...].astype(jnp.bfloat16)
        wg = wg_ref[0].astype(jnp.bfloat16)
        wu = wu_ref[0].astype(jnp.bfloat16)
        wd = wd_ref[0].astype(jnp.bfloat16)
        a = lax.dot_general(xb, wg, (((1,), (1,)), ((), ())),
                            preferred_element_type=jnp.float32)
        u = lax.dot_general(xb, wu, (((1,), (1,)), ((), ())),
                            preferred_element_type=jnp.float32)
        h = (a * jax.nn.sigmoid(a) * u).astype(jnp.bfloat16)
        out_ref[...] = lax.dot_general(h, wd, (((1,), (1,)), ((), ())),
                                       preferred_element_type=jnp.float32)


def _shared_body(x_ref, sg_ref, su_ref, sd_ref, out_ref):
    xb = x_ref[...].astype(jnp.bfloat16)
    a = lax.dot_general(xb, sg_ref[...].astype(jnp.bfloat16),
                        (((1,), (1,)), ((), ())),
                        preferred_element_type=jnp.float32)
    u = lax.dot_general(xb, su_ref[...].astype(jnp.bfloat16),
                        (((1,), (1,)), ((), ())),
                        preferred_element_type=jnp.float32)
    h = (a * jax.nn.sigmoid(a) * u).astype(jnp.bfloat16)
    out_ref[...] = lax.dot_general(h, sd_ref[...].astype(jnp.bfloat16),
                                   (((1,), (1,)), ((), ())),
                                   preferred_element_type=jnp.float32)


def _combine_body(sh_ref, gp_ref, w_ref, out_ref):
    gp = gp_ref[...]
    ge = gp[:, :HB]
    go = gp[:, HB:]
    w = w_ref[...]
    out_ref[...] = sh_ref[...] + w[:, 0:1] * ge + w[:, 1:2] * go


def _sc_dims():
    try:
        info = plsc.get_sparse_core_info()
        return info.num_cores, info.num_subcores
    except Exception:
        return 2, 16


def _make_dispatch(nc, ns):
    nw = nc * ns
    bpw = NPAIR // nw
    ch = 64
    mesh = plsc.VectorSubcoreMesh(core_axis_name="c", subcore_axis_name="s")

    @functools.partial(
        pl.kernel, mesh=mesh,
        out_type=jax.ShapeDtypeStruct((NPAD, HB), jnp.float32),
        scratch_types=[
            pltpu.VMEM((ch,), jnp.int32),
            pltpu.VMEM((ch,), jnp.int32),
            pltpu.VMEM((ch, HB), jnp.float32),
            pltpu.SemaphoreType.DMA,
        ],
    )
    def dispatch(x_hbm, dest_hbm, xs_hbm, src_v, dst_v, rows_v, sem):
        wid = lax.axis_index("s") * nc + lax.axis_index("c")
        base = wid * bpw

        def chunk(c, carry):
            off = base + c * ch
            for v in range(ch // 16):
                io = lax.iota(jnp.int32, 16)
                src_v[pl.ds(v * 16, 16)] = lax.shift_right_logical(
                    off + v * 16 + io, 1)
            pltpu.sync_copy(dest_hbm.at[pl.ds(off, ch)], dst_v)
            pltpu.async_copy(x_hbm.at[src_v], rows_v, sem).wait()
            pltpu.async_copy(rows_v, xs_hbm.at[dst_v], sem).wait()
            return carry

        lax.fori_loop(0, bpw // ch, chunk, 0)

    return dispatch


def _make_gather(nc, ns):
    nw = nc * ns
    bpw = NPAIR // nw
    ch = 32
    mesh = plsc.VectorSubcoreMesh(core_axis_name="c", subcore_axis_name="s")

    @functools.partial(
        pl.kernel, mesh=mesh,
        out_type=jax.ShapeDtypeStruct((NPAIR, HB), jnp.float32),
        scratch_types=[
            pltpu.VMEM((ch,), jnp.int32),
            pltpu.VMEM((ch, HB), jnp.float32),
            pltpu.SemaphoreType.DMA,
        ],
    )
    def gather(ys_hbm, dest_hbm, gp_hbm, idx_v, rows_v, sem):
        wid = lax.axis_index("s") * nc + lax.axis_index("c")
        base = wid * bpw

        def chunk(c, carry):
            off = base + c * ch
            pltpu.sync_copy(dest_hbm.at[pl.ds(off, ch)], idx_v)
            pltpu.async_copy(ys_hbm.at[idx_v], rows_v, sem).wait()
            pltpu.sync_copy(rows_v, gp_hbm.at[pl.ds(off, ch)])
            return carry

        lax.fori_loop(0, bpw // ch, chunk, 0)

    return gather


def kernel(hidden_states, gate_w, gate_b, Wg, Wu, Wd, Sg, Su, Sd):
    bsz, seq, hid = hidden_states.shape
    x = hidden_states.reshape(-1, hid)

    topk_w, dest, te = pl.pallas_call(
        _gate_route_body,
        out_shape=(
            jax.ShapeDtypeStruct((NTOK, KK), jnp.float32),
            jax.ShapeDtypeStruct((NTOK, KK), jnp.int32),
            jax.ShapeDtypeStruct((NT + 1, 1), jnp.int32),
        ),
    )(x, gate_w, gate_b.reshape(1, EE))
    dest_flat = dest.reshape(-1)
    te_flat = te.reshape(-1)

    nc, ns = _sc_dims()
    cb = 256

    sh = pl.pallas_call(
        _shared_body,
        grid=(NTOK // cb,),
        in_specs=[
            pl.BlockSpec((cb, HB), lambda i: (i, 0)),
            pl.BlockSpec((DF, HB), lambda i: (0, 0)),
            pl.BlockSpec((DF, HB), lambda i: (0, 0)),
            pl.BlockSpec((HB, DF), lambda i: (0, 0)),
        ],
        out_specs=pl.BlockSpec((cb, HB), lambda i: (i, 0)),
        out_shape=jax.ShapeDtypeStruct((NTOK, HB), jnp.float32),
    )(x, Sg, Su, Sd)

    xs = _make_dispatch(nc, ns)(x, dest_flat)

    grid_spec = pltpu.PrefetchScalarGridSpec(
        num_scalar_prefetch=1,
        grid=(NT,),
        in_specs=[
            pl.BlockSpec((MT, HB), lambda i, te_r: (i, 0)),
            pl.BlockSpec((1, DF, HB), lambda i, te_r: (te_r[i], 0, 0)),
            pl.BlockSpec((1, DF, HB), lambda i, te_r: (te_r[i], 0, 0)),
            pl.BlockSpec((1, HB, DF), lambda i, te_r: (te_r[i], 0, 0)),
        ],
        out_specs=pl.BlockSpec((MT, HB), lambda i, te_r: (i, 0)),
    )
    ys = pl.pallas_call(
        _moe_body,
        grid_spec=grid_spec,
        out_shape=jax.ShapeDtypeStruct((NPAD, HB), jnp.float32),
    )(te_flat, xs, Wg, Wu, Wd)

    gp = _make_gather(nc, ns)(ys, dest_flat)
    gp2 = gp.reshape(NTOK, KK * HB)

    y = pl.pallas_call(
        _combine_body,
        grid=(NTOK // cb,),
        in_specs=[
            pl.BlockSpec((cb, HB), lambda i: (i, 0)),
            pl.BlockSpec((cb, KK * HB), lambda i: (i, 0)),
            pl.BlockSpec((cb, KK), lambda i: (i, 0)),
        ],
        out_specs=pl.BlockSpec((cb, HB), lambda i: (i, 0)),
        out_shape=jax.ShapeDtypeStruct((NTOK, HB), jnp.float32),
    )(sh, gp2, topk_w)

    return y.reshape(bsz, seq, hid)


# speedup vs baseline: 9.6751x; 1.0710x over previous
import functools

import jax
import jax.numpy as jnp
from jax import lax
from jax.experimental import pallas as pl
from jax.experimental.pallas import tpu as pltpu
from jax.experimental.pallas import tpu_sc as plsc

HB = 1024
EE = 64
KK = 2
NGG = 8
TGG = 4
DF = 512
WSCALE = 2.5
MT = 128
NT = 96
NPAD = NT * MT
NTOK = 2048
NPAIR = NTOK * KK


def _gate_route_body(x_ref, gw_ref, gb_ref, w_out, dest_out, te_out):
    x = x_ref[...]
    gw = gw_ref[...]
    gb = gb_ref[...]
    logits = lax.dot_general(x, gw, (((1,), (1,)), ((), ())),
                             preferred_element_type=jnp.float32)
    scores = jax.nn.sigmoid(logits)
    sfc = scores + gb
    n = x.shape[0]
    neg = jnp.float32(-jnp.inf)
    iota8 = lax.broadcasted_iota(jnp.int32, (n, NGG), 1)

    gparts = []
    for g in range(NGG):
        sub = sfc[:, g * 8:(g + 1) * 8]
        m1 = jnp.max(sub, axis=-1, keepdims=True)
        j1 = jnp.min(jnp.where(sub == m1, iota8, NGG), axis=-1, keepdims=True)
        m2 = jnp.max(jnp.where(iota8 == j1, neg, sub), axis=-1, keepdims=True)
        gparts.append(m1 + m2)
    gs = jnp.concatenate(gparts, axis=1)

    gmask = jnp.zeros((n, NGG), jnp.float32)
    cur = gs
    for _ in range(TGG):
        m = jnp.max(cur, axis=-1, keepdims=True)
        j = jnp.min(jnp.where(cur == m, iota8, NGG), axis=-1, keepdims=True)
        hit = iota8 == j
        gmask = gmask + hit.astype(jnp.float32)
        cur = jnp.where(hit, neg, cur)

    er = lax.broadcasted_iota(jnp.int32, (NGG, EE), 0)
    ec = lax.broadcasted_iota(jnp.int32, (NGG, EE), 1)
    expand = (er == ec // NGG).astype(jnp.float32)
    smask = lax.dot_general(gmask, expand, (((1,), (0,)), ((), ())),
                            preferred_element_type=jnp.float32)

    iota64 = lax.broadcasted_iota(jnp.int32, (n, EE), 1)
    tmp = jnp.where(smask > 0, sfc, neg)
    ohs, ws = [], []
    for _ in range(KK):
        m = jnp.max(tmp, axis=-1, keepdims=True)
        j = jnp.min(jnp.where(tmp == m, iota64, EE), axis=-1, keepdims=True)
        oh = (iota64 == j).astype(jnp.float32)
        ohs.append(oh)
        ws.append(jnp.sum(oh * scores, axis=-1, keepdims=True))
        tmp = jnp.where(oh > 0, neg, tmp)
    w0, w1 = ws
    denom = w0 + w1 + 1e-20
    w_out[...] = jnp.concatenate(
        [w0 / denom * WSCALE, w1 / denom * WSCALE], axis=1)

    oh_all = ohs[0] + ohs[1]
    counts = jnp.sum(oh_all, axis=0, keepdims=True)
    ptiles = jnp.floor((counts + (MT - 1)) * (1.0 / MT))
    ur = lax.broadcasted_iota(jnp.int32, (EE, EE), 0)
    uc = lax.broadcasted_iota(jnp.int32, (EE, EE), 1)
    strict_u = (ur < uc).astype(jnp.float32)
    tstart = lax.dot_general(ptiles, strict_u, (((1,), (0,)), ((), ())),
                             preferred_element_type=jnp.float32)
    tstart_i = tstart.astype(jnp.int32)
    trow = lax.broadcasted_iota(jnp.int32, (NT + 1, EE), 0)
    te = jnp.sum((jnp.broadcast_to(tstart_i, (NT + 1, EE)) <= trow)
                 .astype(jnp.int32), axis=-1, keepdims=True) - 1
    used = jnp.sum(ptiles, axis=-1, keepdims=True).astype(jnp.int32)
    te_out[...] = jnp.concatenate([te[:NT], used], axis=0)

    blk = 256
    br = lax.broadcasted_iota(jnp.int32, (blk, blk), 0)
    bc = lax.broadcasted_iota(jnp.int32, (blk, blk), 1)
    lstrict = (bc < br).astype(jnp.float32)
    tot = jnp.zeros((1, EE), jnp.float32)
    cparts = []
    for b in range(n // blk):
        ohb = oh_all[b * blk:(b + 1) * blk, :]
        cb = lax.dot_general(lstrict, ohb, (((1,), (0,)), ((), ())),
                             preferred_element_type=jnp.float32) + tot
        cparts.append(cb)
        tot = tot + jnp.sum(ohb, axis=0, keepdims=True)
    cexc = jnp.concatenate(cparts, axis=0)

    rowbase = tstart * MT + cexc
    d0 = jnp.sum(ohs[0] * rowbase, axis=-1, keepdims=True)
    d1 = jnp.sum(ohs[1] * rowbase, axis=-1, keepdims=True)
    dest_out[...] = jnp.concatenate([d0, d1], axis=1).astype(jnp.int32)


def _moe_body(te_ref, xs_ref, wg_ref, wu_ref, wd_ref, out_ref):
    @pl.when(pl.program_id(0) < te_ref[NT])
    def _():
        xb = xs_ref[...].astype(jnp.bfloat16)
        wg = wg_ref[0].astype(jnp.bfloat16)
        wu = wu_ref[0].astype(jnp.bfloat16)
        wd = wd_ref[0].astype(jnp.bfloat16)
        a = lax.dot_general(xb, wg, (((1,), (1,)), ((), ())),
                            preferred_element_type=jnp.float32)
        u = lax.dot_general(xb, wu, (((1,), (1,)), ((), ())),
                            preferred_element_type=jnp.float32)
        h = (a * jax.nn.sigmoid(a) * u).astype(jnp.bfloat16)
        out_ref[...] = lax.dot_general(h, wd, (((1,), (1,)), ((), ())),
                                       preferred_element_type=jnp.float32)


def _shared_body(x_ref, sg_ref, su_ref, sd_ref, out_ref):
    xb = x_ref[...].astype(jnp.bfloat16)
    a = lax.dot_general(xb, sg_ref[...].astype(jnp.bfloat16),
                        (((1,), (1,)), ((), ())),
                        preferred_element_type=jnp.float32)
    u = lax.dot_general(xb, su_ref[...].astype(jnp.bfloat16),
                        (((1,), (1,)), ((), ())),
                        preferred_element_type=jnp.float32)
    h = (a * jax.nn.sigmoid(a) * u).astype(jnp.bfloat16)
    out_ref[...] = lax.dot_general(h, sd_ref[...].astype(jnp.bfloat16),
                                   (((1,), (1,)), ((), ())),
                                   preferred_element_type=jnp.float32)


def _combine_body(sh_ref, gp_ref, w_ref, out_ref):
    gp = gp_ref[...]
    ge = gp[:, :HB]
    go = gp[:, HB:]
    w = w_ref[...]
    out_ref[...] = sh_ref[...] + w[:, 0:1] * ge + w[:, 1:2] * go


def _sc_dims():
    try:
        info = plsc.get_sparse_core_info()
        return info.num_cores, info.num_subcores
    except Exception:
        return 2, 16


def _make_dispatch(nc, ns):
    nw = nc * ns
    bpw = NPAIR // nw
    ch = 64
    mesh = plsc.VectorSubcoreMesh(core_axis_name="c", subcore_axis_name="s")

    @functools.partial(
        pl.kernel, mesh=mesh,
        out_type=jax.ShapeDtypeStruct((NPAD, HB), jnp.float32),
        scratch_types=[
            pltpu.VMEM((ch,), jnp.int32),
            pltpu.VMEM((ch,), jnp.int32),
            pltpu.VMEM((ch, HB), jnp.float32),
            pltpu.SemaphoreType.DMA,
        ],
    )
    def dispatch(x_hbm, dest_hbm, xs_hbm, src_v, dst_v, rows_v, sem):
        wid = lax.axis_index("s") * nc + lax.axis_index("c")
        base = wid * bpw

        def chunk(c, carry):
            off = base + c * ch
            for v in range(ch // 16):
                io = lax.iota(jnp.int32, 16)
                src_v[pl.ds(v * 16, 16)] = lax.shift_right_logical(
                    off + v * 16 + io, 1)
            pltpu.sync_copy(dest_hbm.at[pl.ds(off, ch)], dst_v)
            pltpu.async_copy(x_hbm.at[src_v], rows_v, sem).wait()
            pltpu.async_copy(rows_v, xs_hbm.at[dst_v], sem).wait()
            return carry

        lax.fori_loop(0, bpw // ch, chunk, 0)

    return dispatch


def _make_gather(nc, ns):
    nw = nc * ns
    bpw = NPAIR // nw
    ch = 32
    mesh = plsc.VectorSubcoreMesh(core_axis_name="c", subcore_axis_name="s")

    @functools.partial(
        pl.kernel, mesh=mesh,
        out_type=jax.ShapeDtypeStruct((NPAIR, HB), jnp.float32),
        scratch_types=[
            pltpu.VMEM((ch,), jnp.int32),
            pltpu.VMEM((ch, HB), jnp.float32),
            pltpu.SemaphoreType.DMA,
        ],
    )
    def gather(ys_hbm, dest_hbm, gp_hbm, idx_v, rows_v, sem):
        wid = lax.axis_index("s") * nc + lax.axis_index("c")
        base = wid * bpw

        def chunk(c, carry):
            off = base + c * ch
            pltpu.sync_copy(dest_hbm.at[pl.ds(off, ch)], idx_v)
            pltpu.async_copy(ys_hbm.at[idx_v], rows_v, sem).wait()
            pltpu.sync_copy(rows_v, gp_hbm.at[pl.ds(off, ch)])
            return carry

        lax.fori_loop(0, bpw // ch, chunk, 0)

    return gather


def kernel(hidden_states, gate_w, gate_b, Wg, Wu, Wd, Sg, Su, Sd):
    bsz, seq, hid = hidden_states.shape
    x = hidden_states.reshape(-1, hid)

    topk_w, dest, te = pl.pallas_call(
        _gate_route_body,
        out_shape=(
            jax.ShapeDtypeStruct((NTOK, KK), jnp.float32),
            jax.ShapeDtypeStruct((NTOK, KK), jnp.int32),
            jax.ShapeDtypeStruct((NT + 1, 1), jnp.int32),
        ),
    )(x, gate_w, gate_b.reshape(1, EE))
    dest_flat = dest.reshape(-1)
    te_flat = te.reshape(-1)

    nc, ns = _sc_dims()
    cb = 256

    sh = pl.pallas_call(
        _shared_body,
        grid=(NTOK // cb,),
        in_specs=[
            pl.BlockSpec((cb, HB), lambda i: (i, 0)),
            pl.BlockSpec((DF, HB), lambda i: (0, 0)),
            pl.BlockSpec((DF, HB), lambda i: (0, 0)),
            pl.BlockSpec((HB, DF), lambda i: (0, 0)),
        ],
        out_specs=pl.BlockSpec((cb, HB), lambda i: (i, 0)),
        out_shape=jax.ShapeDtypeStruct((NTOK, HB), jnp.float32),
    )(x, Sg, Su, Sd)

    xs = _make_dispatch(nc, ns)(x, dest_flat)

    grid_spec = pltpu.PrefetchScalarGridSpec(
        num_scalar_prefetch=1,
        grid=(NT,),
        in_specs=[
            pl.BlockSpec((MT, HB),
                         lambda i, te_r: (jnp.minimum(i, te_r[NT] - 1), 0)),
            pl.BlockSpec((1, DF, HB), lambda i, te_r: (te_r[i], 0, 0)),
            pl.BlockSpec((1, DF, HB), lambda i, te_r: (te_r[i], 0, 0)),
            pl.BlockSpec((1, HB, DF), lambda i, te_r: (te_r[i], 0, 0)),
        ],
        out_specs=pl.BlockSpec(
            (MT, HB), lambda i, te_r: (jnp.minimum(i, te_r[NT] - 1), 0)),
    )
    ys = pl.pallas_call(
        _moe_body,
        grid_spec=grid_spec,
        out_shape=jax.ShapeDtypeStruct((NPAD, HB), jnp.float32),
    )(te_flat, xs, Wg, Wu, Wd)

    gp = _make_gather(nc, ns)(ys, dest_flat)
    gp2 = gp.reshape(NTOK, KK * HB)

    y = pl.pallas_call(
        _combine_body,
        grid=(NTOK // cb,),
        in_specs=[
            pl.BlockSpec((cb, HB), lambda i: (i, 0)),
            pl.BlockSpec((cb, KK * HB), lambda i: (i, 0)),
            pl.BlockSpec((cb, KK), lambda i: (i, 0)),
        ],
        out_specs=pl.BlockSpec((cb, HB), lambda i: (i, 0)),
        out_shape=jax.ShapeDtypeStruct((NTOK, HB), jnp.float32),
    )(sh, gp2, topk_w)

    return y.reshape(bsz, seq, hid)
